# Initial kernel scaffold; baseline (speedup 1.0000x reference)
import jax
import jax.numpy as jnp
from jax.experimental import pallas as pl


def kernel(src_tokens, tgt_tokens, src_table, tgt_table):
    raise NotImplementedError("write your pallas kernel here")


# speedup vs baseline: 4.2889x; 4.2889x over previous
import functools

import jax
import jax.numpy as jnp
from jax import lax
from jax.experimental import pallas as pl
from jax.experimental.pallas import tpu as pltpu
from jax.experimental.pallas import tpu_sc as plsc

B = 4096
L = 50
EMB = 64
N = B * L
CHUNK = 128
NC, NS = 2, 16
NW = NC * NS
PER_W = N // NW
NCHUNK = PER_W // CHUNK


def _emb_kernel(src_tbl, tgt_tbl, src_idx, tgt_idx, out,
                idx_v, rows_v, sem, osem):
    wid = lax.axis_index("s") * NC + lax.axis_index("c")
    row_base = wid * PER_W

    for t, (tbl, idx_hbm) in enumerate(((src_tbl, src_idx), (tgt_tbl, tgt_idx))):
        pltpu.sync_copy(idx_hbm.at[pl.ds(row_base, PER_W)], idx_v)
        out_base = t * N + row_base

        @pl.loop(0, NCHUNK)
        def _(j):
            pltpu.async_copy(tbl.at[idx_v.at[pl.ds(j * CHUNK, CHUNK)]],
                             rows_v, sem).wait()
            pltpu.async_copy(rows_v, out.at[pl.ds(out_base + j * CHUNK, CHUNK)],
                             osem).wait()


@jax.jit
def kernel(src_tokens, tgt_tokens, src_table, tgt_table):
    src_idx = src_tokens.reshape(N).astype(jnp.int32)
    tgt_idx = tgt_tokens.reshape(N).astype(jnp.int32)

    mesh = plsc.VectorSubcoreMesh(core_axis_name="c", subcore_axis_name="s")
    out = pl.kernel(
        _emb_kernel,
        out_type=jax.ShapeDtypeStruct((2 * N, EMB), jnp.float32),
        mesh=mesh,
        scratch_types=[
            pltpu.VMEM((PER_W,), jnp.int32),
            pltpu.VMEM((CHUNK, EMB), jnp.float32),
            pltpu.SemaphoreType.DMA,
            pltpu.SemaphoreType.DMA,
        ],
        compiler_params=pltpu.CompilerParams(use_tc_tiling_on_sc=False),
    )(src_table, tgt_table, src_idx, tgt_idx)
    return out.reshape(2, B, L, EMB)


# speedup vs baseline: 4.9627x; 1.1571x over previous
import jax
import jax.numpy as jnp
from jax import lax
from jax.experimental import pallas as pl
from jax.experimental.pallas import tpu as pltpu
from jax.experimental.pallas import tpu_sc as plsc

B = 4096
L = 50
EMB = 64
N = B * L
CHUNK = 128
NC, NS = 2, 16
NW = NC * NS
PER_W = N // NW
NCHUNK = PER_W // CHUNK
NBUF = 5
AHEAD = 2


def _emb_kernel(src_tbl, tgt_tbl, src_idx, tgt_idx, out, idx_v, *scratch):
    rows = scratch[:NBUF]
    gsem = scratch[NBUF:2 * NBUF]
    wsem = scratch[2 * NBUF:]
    wid = lax.axis_index("s") * NC + lax.axis_index("c")
    row_base = wid * PER_W

    for t, (tbl, idx_hbm) in enumerate(((src_tbl, src_idx), (tgt_tbl, tgt_idx))):
        pltpu.sync_copy(idx_hbm.at[pl.ds(row_base, PER_W)], idx_v)
        out_base = t * N + row_base

        def gather(j, b):
            pltpu.async_copy(tbl.at[idx_v.at[pl.ds(j * CHUNK, CHUNK)]],
                             rows[b], gsem[b])

        def gather_wait(j, b):
            pltpu.make_async_copy(tbl.at[idx_v.at[pl.ds(j * CHUNK, CHUNK)]],
                                  rows[b], gsem[b]).wait()

        def wb(j, b):
            pltpu.async_copy(rows[b], out.at[pl.ds(out_base + j * CHUNK, CHUNK)],
                             wsem[b])

        def wb_wait(j, b):
            pltpu.make_async_copy(rows[b],
                                  out.at[pl.ds(out_base + j * CHUNK, CHUNK)],
                                  wsem[b]).wait()

        for b in range(AHEAD):
            gather(b, b)

        @pl.loop(0, NCHUNK, step=NBUF)
        def _(j0):
            for b in range(NBUF):
                j = j0 + b
                nxt = (b + AHEAD) % NBUF

                @pl.when(j >= NBUF - AHEAD)
                def _():
                    wb_wait(j + AHEAD - NBUF, nxt)

                @pl.when(j < NCHUNK - AHEAD)
                def _():
                    gather(j + AHEAD, nxt)

                gather_wait(j, b)
                wb(j, b)

        for j in range(NCHUNK - (NBUF - AHEAD), NCHUNK):
            wb_wait(j, j % NBUF)


@jax.jit
def kernel(src_tokens, tgt_tokens, src_table, tgt_table):
    src_idx = src_tokens.reshape(N).astype(jnp.int32)
    tgt_idx = tgt_tokens.reshape(N).astype(jnp.int32)

    mesh = plsc.VectorSubcoreMesh(core_axis_name="c", subcore_axis_name="s")
    out = pl.kernel(
        _emb_kernel,
        out_type=jax.ShapeDtypeStruct((2 * N, EMB), jnp.float32),
        mesh=mesh,
        scratch_types=(
            [pltpu.VMEM((PER_W,), jnp.int32)]
            + [pltpu.VMEM((CHUNK, EMB), jnp.float32) for _ in range(NBUF)]
            + [pltpu.SemaphoreType.DMA for _ in range(2 * NBUF)]
        ),
        compiler_params=pltpu.CompilerParams(use_tc_tiling_on_sc=False),
    )(src_table, tgt_table, src_idx, tgt_idx)
    return out.reshape(2, B, L, EMB)


# speedup vs baseline: 4.9698x; 1.0014x over previous
import jax
import jax.numpy as jnp
from jax import lax
from jax.experimental import pallas as pl
from jax.experimental.pallas import tpu as pltpu
from jax.experimental.pallas import tpu_sc as plsc

B = 4096
L = 50
EMB = 64
N = B * L
CHUNK = 128
NC, NS = 2, 16
NW = NC * NS
PER_W = N // NW
NCHUNK = PER_W // CHUNK
NBUF = 10
AHEAD = 5


def _emb_kernel(src_tbl, tgt_tbl, src_idx, tgt_idx, out, idx_v, *scratch):
    rows = scratch[:NBUF]
    gsem = scratch[NBUF:2 * NBUF]
    wsem = scratch[2 * NBUF:]
    wid = lax.axis_index("s") * NC + lax.axis_index("c")
    row_base = wid * PER_W

    for t, (tbl, idx_hbm) in enumerate(((src_tbl, src_idx), (tgt_tbl, tgt_idx))):
        pltpu.sync_copy(idx_hbm.at[pl.ds(row_base, PER_W)], idx_v)
        out_base = t * N + row_base

        def gather(j, b):
            pltpu.async_copy(tbl.at[idx_v.at[pl.ds(j * CHUNK, CHUNK)]],
                             rows[b], gsem[b])

        def gather_wait(j, b):
            pltpu.make_async_copy(tbl.at[idx_v.at[pl.ds(j * CHUNK, CHUNK)]],
                                  rows[b], gsem[b]).wait()

        def wb(j, b):
            pltpu.async_copy(rows[b], out.at[pl.ds(out_base + j * CHUNK, CHUNK)],
                             wsem[b])

        def wb_wait(j, b):
            pltpu.make_async_copy(rows[b],
                                  out.at[pl.ds(out_base + j * CHUNK, CHUNK)],
                                  wsem[b]).wait()

        for b in range(AHEAD):
            gather(b, b)

        @pl.loop(0, NCHUNK, step=NBUF)
        def _(j0):
            for b in range(NBUF):
                j = j0 + b
                nxt = (b + AHEAD) % NBUF

                @pl.when(j >= NBUF - AHEAD)
                def _():
                    wb_wait(j + AHEAD - NBUF, nxt)

                @pl.when(j < NCHUNK - AHEAD)
                def _():
                    gather(j + AHEAD, nxt)

                gather_wait(j, b)
                wb(j, b)

        for j in range(NCHUNK - (NBUF - AHEAD), NCHUNK):
            wb_wait(j, j % NBUF)


@jax.jit
def kernel(src_tokens, tgt_tokens, src_table, tgt_table):
    src_idx = src_tokens.reshape(N).astype(jnp.int32)
    tgt_idx = tgt_tokens.reshape(N).astype(jnp.int32)

    mesh = plsc.VectorSubcoreMesh(core_axis_name="c", subcore_axis_name="s")
    out = pl.kernel(
        _emb_kernel,
        out_type=jax.ShapeDtypeStruct((2 * N, EMB), jnp.float32),
        mesh=mesh,
        scratch_types=(
            [pltpu.VMEM((PER_W,), jnp.int32)]
            + [pltpu.VMEM((CHUNK, EMB), jnp.float32) for _ in range(NBUF)]
            + [pltpu.SemaphoreType.DMA for _ in range(2 * NBUF)]
        ),
        compiler_params=pltpu.CompilerParams(use_tc_tiling_on_sc=False),
    )(src_table, tgt_table, src_idx, tgt_idx)
    return out.reshape(2, B, L, EMB)
